# Initial kernel scaffold; baseline (speedup 1.0000x reference)
import jax
import jax.numpy as jnp
from jax.experimental import pallas as pl


def kernel(idx, weight):
    raise NotImplementedError("write your pallas kernel here")


# speedup vs baseline: 1.5578x; 1.5578x over previous
import functools

import jax
import jax.numpy as jnp
from jax import lax
from jax.experimental import pallas as pl
from jax.experimental.pallas import tpu as pltpu
from jax.experimental.pallas import tpu_sc as plsc

NC = 2
NS = 16
NW = NC * NS

IDXW = 128
GRP = 8

D = 32


def _build(B):
    assert B % (NW * GRP * IDXW) == 0
    b_per_w = B // NW
    steps = b_per_w // IDXW
    n_grp = steps // GRP
    mesh = plsc.VectorSubcoreMesh(core_axis_name="c", subcore_axis_name="s")

    @functools.partial(
        pl.kernel,
        mesh=mesh,
        out_type=jax.ShapeDtypeStruct((B, D), jnp.float32),
        scratch_types=[
            pltpu.VMEM((steps, IDXW), jnp.int32),
            pltpu.VMEM((GRP * IDXW, D), jnp.float32),
            pltpu.SemaphoreType.DMA,
        ],
        compiler_params=pltpu.CompilerParams(use_tc_tiling_on_sc=False),
    )
    def emb(idx_hbm, table_hbm, out_hbm, idx_v, rows_v, sem):
        wid = lax.axis_index("s") * NC + lax.axis_index("c")
        base = wid * b_per_w
        pltpu.sync_copy(idx_hbm.at[wid], idx_v)

        def body(g, _):
            copies = [
                pltpu.async_copy(
                    table_hbm.at[idx_v.at[g * GRP + j]],
                    rows_v.at[pl.ds(j * IDXW, IDXW)],
                    sem,
                )
                for j in range(GRP)
            ]
            for c in copies:
                c.wait()
            pltpu.sync_copy(
                rows_v, out_hbm.at[pl.ds(base + g * (GRP * IDXW), GRP * IDXW)]
            )
            return ()

        lax.fori_loop(0, n_grp, body, ())

    return emb


def kernel(idx, weight):
    B = idx.size
    idx3 = idx.reshape(NW, B // (NW * IDXW), IDXW).astype(jnp.int32)
    out = _build(B)(idx3, weight)
    return out.reshape(idx.shape + (weight.shape[1],))


# speedup vs baseline: 1.5754x; 1.0113x over previous
import functools

import jax
import jax.numpy as jnp
from jax import lax
from jax.experimental import pallas as pl
from jax.experimental.pallas import tpu as pltpu
from jax.experimental.pallas import tpu_sc as plsc

NC = 2
NS = 16
NW = NC * NS

IDXW = 128
GRP = 8
GROWS = GRP * IDXW

D = 32


def _build(B):
    assert B % (NW * GROWS) == 0
    b_per_w = B // NW
    steps = b_per_w // IDXW
    n_grp = steps // GRP
    mesh = plsc.VectorSubcoreMesh(core_axis_name="c", subcore_axis_name="s")

    @functools.partial(
        pl.kernel,
        mesh=mesh,
        out_type=jax.ShapeDtypeStruct((B, D), jnp.float32),
        scratch_types=[
            pltpu.VMEM((steps, IDXW), jnp.int32),
            pltpu.VMEM((2, GROWS, D), jnp.float32),
            pltpu.SemaphoreType.DMA,
            pltpu.SemaphoreType.DMA,
            pltpu.SemaphoreType.DMA,
        ],
        compiler_params=pltpu.CompilerParams(use_tc_tiling_on_sc=False),
    )
    def emb(idx_hbm, table_hbm, out_hbm, idx_v, rows_v, sem0, sem1, sem_w):
        wid = lax.axis_index("s") * NC + lax.axis_index("c")
        base = wid * b_per_w
        pltpu.sync_copy(idx_hbm.at[wid], idx_v)

        def fire(g, b, sem):
            for j in range(GRP):
                pltpu.async_copy(
                    table_hbm.at[idx_v.at[g * GRP + j]],
                    rows_v.at[b].at[pl.ds(j * IDXW, IDXW)],
                    sem,
                )

        def drain(g, b, sem):
            for j in range(GRP):
                pltpu.make_async_copy(
                    table_hbm.at[idx_v.at[g * GRP + j]],
                    rows_v.at[b].at[pl.ds(j * IDXW, IDXW)],
                    sem,
                ).wait()

        def out_copy(g, b):
            return pltpu.make_async_copy(
                rows_v.at[b], out_hbm.at[pl.ds(base + g * GROWS, GROWS)], sem_w
            )

        fire(0, 0, sem0)

        def body(g, _):
            b = lax.rem(g, 2)

            @pl.when(g > 0)
            def _():
                out_copy(g - 1, 1 - b).wait()

            @pl.when(g < n_grp - 1)
            def _():
                @pl.when(b == 0)
                def _():
                    fire(g + 1, 1, sem1)

                @pl.when(b == 1)
                def _():
                    fire(g + 1, 0, sem0)

            @pl.when(b == 0)
            def _():
                drain(g, 0, sem0)

            @pl.when(b == 1)
            def _():
                drain(g, 1, sem1)

            out_copy(g, b).start()
            return ()

        lax.fori_loop(0, n_grp, body, ())
        out_copy(n_grp - 1, lax.rem(n_grp - 1, 2)).wait()

    return emb


def kernel(idx, weight):
    B = idx.size
    idx3 = idx.reshape(NW, B // (NW * IDXW), IDXW).astype(jnp.int32)
    out = _build(B)(idx3, weight)
    return out.reshape(idx.shape + (weight.shape[1],))


# speedup vs baseline: 1.5760x; 1.0004x over previous
import functools

import jax
import jax.numpy as jnp
from jax import lax
from jax.experimental import pallas as pl
from jax.experimental.pallas import tpu as pltpu
from jax.experimental.pallas import tpu_sc as plsc

NC = 2
NS = 16
NW = NC * NS

IDXW = 1664
GRP = 1
GROWS = GRP * IDXW

D = 32


def _build(B):
    assert B % (NW * GROWS) == 0
    b_per_w = B // NW
    steps = b_per_w // IDXW
    n_grp = steps // GRP
    mesh = plsc.VectorSubcoreMesh(core_axis_name="c", subcore_axis_name="s")

    @functools.partial(
        pl.kernel,
        mesh=mesh,
        out_type=jax.ShapeDtypeStruct((B, D), jnp.float32),
        scratch_types=[
            pltpu.VMEM((steps, IDXW), jnp.int32),
            pltpu.VMEM((2, GROWS, D), jnp.float32),
            pltpu.SemaphoreType.DMA,
            pltpu.SemaphoreType.DMA,
            pltpu.SemaphoreType.DMA,
        ],
        compiler_params=pltpu.CompilerParams(use_tc_tiling_on_sc=False),
    )
    def emb(idx_hbm, table_hbm, out_hbm, idx_v, rows_v, sem0, sem1, sem_w):
        wid = lax.axis_index("s") * NC + lax.axis_index("c")
        base = wid * b_per_w
        pltpu.sync_copy(idx_hbm.at[wid], idx_v)

        def fire(g, b, sem):
            for j in range(GRP):
                pltpu.async_copy(
                    table_hbm.at[idx_v.at[g * GRP + j]],
                    rows_v.at[b].at[pl.ds(j * IDXW, IDXW)],
                    sem,
                )

        def drain(g, b, sem):
            for j in range(GRP):
                pltpu.make_async_copy(
                    table_hbm.at[idx_v.at[g * GRP + j]],
                    rows_v.at[b].at[pl.ds(j * IDXW, IDXW)],
                    sem,
                ).wait()

        def out_copy(g, b):
            return pltpu.make_async_copy(
                rows_v.at[b], out_hbm.at[pl.ds(base + g * GROWS, GROWS)], sem_w
            )

        fire(0, 0, sem0)

        def body(g, _):
            b = lax.rem(g, 2)

            @pl.when(g > 0)
            def _():
                out_copy(g - 1, 1 - b).wait()

            @pl.when(g < n_grp - 1)
            def _():
                @pl.when(b == 0)
                def _():
                    fire(g + 1, 1, sem1)

                @pl.when(b == 1)
                def _():
                    fire(g + 1, 0, sem0)

            @pl.when(b == 0)
            def _():
                drain(g, 0, sem0)

            @pl.when(b == 1)
            def _():
                drain(g, 1, sem1)

            out_copy(g, b).start()
            return ()

        lax.fori_loop(0, n_grp, body, ())
        out_copy(n_grp - 1, lax.rem(n_grp - 1, 2)).wait()

    return emb


def kernel(idx, weight):
    B = idx.size
    idx3 = idx.reshape(NW, B // (NW * IDXW), IDXW).astype(jnp.int32)
    out = _build(B)(idx3, weight)
    return out.reshape(idx.shape + (weight.shape[1],))
